# baseline (speedup 1.0000x reference)
import functools

import jax
import jax.numpy as jnp
from jax import lax
from jax.experimental import pallas as pl
from jax.experimental.pallas import tpu as pltpu
from jax.experimental.pallas import tpu_sc as plsc

N = 100000
E = 3200000
F = 128
H = 16
C = 40

NC = 2
NS = 16
NW = NC * NS

CH = 128
RW = 816
ROWS = NW * RW
EPAD = ROWS * CH
K = 16
DEPTH = 4
NBUF = 8

NACC = 100352
RT = NACC // NS


def _sc_body(with_counts, *refs):
    if with_counts:
        (src_hbm, dst_hbm, tab_hbm, z16_hbm, z1_hbm,
         out_hbm, cnt_out_hbm,
         acc_sh, cnt_sh, sidx, didx, rows_v, ones_v, z1v,
         *sems) = refs
    else:
        (src_hbm, dst_hbm, tab_hbm, z16_hbm,
         out_hbm,
         acc_sh, sidx, didx, rows_v,
         *sems) = refs
    sem_g = sems[:NBUF]
    sem_s = sems[NBUF:2 * NBUF]
    sem_c = sems[2 * NBUF] if with_counts else None

    core = lax.axis_index("c")
    sub = lax.axis_index("s")
    w = sub * NC + core

    r0 = sub * RT

    def zinit(k, carry):
        o = r0 + k * CH
        pltpu.sync_copy(z16_hbm.at[pl.ds(o, CH)], rows_v.at[0])
        pltpu.sync_copy(rows_v.at[0], acc_sh.at[pl.ds(o, CH)])
        if with_counts:
            pltpu.sync_copy(z1_hbm.at[pl.ds(o, CH)], z1v)
            pltpu.sync_copy(z1v, cnt_sh.at[pl.ds(o, CH)])
        return carry

    lax.fori_loop(0, RT // CH, zinit, 0)
    if with_counts:
        for i in range(CH // 16):
            ones_v[pl.ds(i * 16, 16)] = jnp.ones((16,), jnp.float32)
    plsc.subcore_barrier()

    base_row = w * RW

    def step(i, carry):
        row = base_row + i * K
        pltpu.sync_copy(src_hbm.at[pl.ds(row, K)], sidx)
        pltpu.sync_copy(dst_hbm.at[pl.ds(row, K)], didx)
        dg = {}
        dsc = {}
        dcnt = []
        for u in range(DEPTH):
            dg[u] = pltpu.async_copy(tab_hbm.at[sidx.at[u]], rows_v.at[u],
                                     sem_g[u])
        for u in range(K):
            dg[u].wait()
            dsc[u] = pltpu.async_copy(rows_v.at[u % NBUF],
                                      acc_sh.at[didx.at[u]],
                                      sem_s[u % NBUF], add=True)
            if with_counts:
                dcnt.append(pltpu.async_copy(ones_v, cnt_sh.at[didx.at[u]],
                                             sem_c, add=True))
            nu = u + DEPTH
            if nu < K:
                if nu >= NBUF:
                    dsc[nu - NBUF].wait()
                dg[nu] = pltpu.async_copy(tab_hbm.at[sidx.at[nu]],
                                          rows_v.at[nu % NBUF],
                                          sem_g[nu % NBUF])
        for u in range(K - NBUF, K):
            dsc[u].wait()
        for d in dcnt:
            d.wait()
        return carry

    lax.fori_loop(0, RW // K, step, 0)

    plsc.subcore_barrier()

    def cout(k, carry):
        o = r0 + k * CH
        pltpu.sync_copy(acc_sh.at[pl.ds(o, CH)], rows_v.at[0])
        pltpu.sync_copy(rows_v.at[0], out_hbm.at[core, pl.ds(o, CH)])
        if with_counts:
            pltpu.sync_copy(cnt_sh.at[pl.ds(o, CH)], z1v)
            pltpu.sync_copy(z1v, cnt_out_hbm.at[pl.ds(core * NACC + o, CH)])
        return carry

    lax.fori_loop(0, RT // CH, cout, 0)


def _make_sc_pass(with_counts):
    mesh = plsc.VectorSubcoreMesh(core_axis_name="c", subcore_axis_name="s",
                                  num_cores=NC, num_subcores=NS)
    if with_counts:
        out_type = (jax.ShapeDtypeStruct((NC, NACC, H), jnp.float32),
                    jax.ShapeDtypeStruct((NC * NACC,), jnp.float32))
        scratch = [
            pltpu.VMEM_SHARED((NACC, H), jnp.float32),
            pltpu.VMEM_SHARED((NACC,), jnp.float32),
            pltpu.VMEM((K, CH), jnp.int32),
            pltpu.VMEM((K, CH), jnp.int32),
            pltpu.VMEM((NBUF, CH, H), jnp.float32),
            pltpu.VMEM((CH,), jnp.float32),
            pltpu.VMEM((CH,), jnp.float32),
        ] + [pltpu.SemaphoreType.DMA] * (2 * NBUF + 1)
    else:
        out_type = jax.ShapeDtypeStruct((NC, NACC, H), jnp.float32)
        scratch = [
            pltpu.VMEM_SHARED((NACC, H), jnp.float32),
            pltpu.VMEM((K, CH), jnp.int32),
            pltpu.VMEM((K, CH), jnp.int32),
            pltpu.VMEM((NBUF, CH, H), jnp.float32),
        ] + [pltpu.SemaphoreType.DMA] * (2 * NBUF)
    return pl.kernel(functools.partial(_sc_body, with_counts),
                     out_type=out_type, mesh=mesh, scratch_types=scratch,
                     compiler_params=pltpu.CompilerParams(
                         use_tc_tiling_on_sc=False))


_sc_pass_counts = _make_sc_pass(True)
_sc_pass = _make_sc_pass(False)


NPK = NACC // 8
ROWB = 2048
GRID1 = NACC // ROWB
PKB = ROWB // 8


def _tc1_body(x_ref, w_ref, b_ref, o_ref):
    o_ref[...] = jnp.dot(x_ref[...], w_ref[...],
                         preferred_element_type=jnp.float32) + b_ref[...]


def _tc1(x, wt, b):
    return pl.pallas_call(
        _tc1_body,
        grid=(GRID1,),
        in_specs=[
            pl.BlockSpec((ROWB, F), lambda i: (i, 0)),
            pl.BlockSpec((F, H), lambda i: (0, 0)),
            pl.BlockSpec((1, H), lambda i: (0, 0)),
        ],
        out_specs=pl.BlockSpec((ROWB, H), lambda i: (i, 0)),
        out_shape=jax.ShapeDtypeStruct((NACC, H), jnp.float32),
    )(x, wt, b)


def _tc2_body(p_ref, c_ref, g_ref):
    p = p_ref[...]
    s = (p[0] + p[1]) / c_ref[...]
    g_ref[...] = jnp.maximum(s, 0.0)


def _tc2(partp, cbp):
    return pl.pallas_call(
        _tc2_body,
        grid=(GRID1,),
        in_specs=[
            pl.BlockSpec((NC, PKB, 128), lambda i: (0, i, 0)),
            pl.BlockSpec((PKB, 128), lambda i: (i, 0)),
        ],
        out_specs=pl.BlockSpec((PKB, 128), lambda i: (i, 0)),
        out_shape=jax.ShapeDtypeStruct((NPK, 128), jnp.float32),
    )(partp, cbp)


def _tc3a_body(p_ref, c_ref, w_ref, b_ref, o_ref):
    p = p_ref[...]
    sp = (p[0] + p[1]) / c_ref[...]
    o_ref[...] = jnp.dot(sp, w_ref[...],
                         preferred_element_type=jnp.float32) + b_ref[...]


def _tc3a(partp, cbp, wd, bd):
    return pl.pallas_call(
        _tc3a_body,
        grid=(GRID1,),
        in_specs=[
            pl.BlockSpec((NC, PKB, 128), lambda i: (0, i, 0)),
            pl.BlockSpec((PKB, 128), lambda i: (i, 0)),
            pl.BlockSpec((128, 8 * C), lambda i: (0, 0)),
            pl.BlockSpec((1, 8 * C), lambda i: (0, 0)),
        ],
        out_specs=pl.BlockSpec((PKB, 8 * C), lambda i: (i, 0)),
        out_shape=jax.ShapeDtypeStruct((NPK, 8 * C), jnp.float32),
    )(partp, cbp, wd, bd)


def _tc3b_body(z_ref, o_ref):
    z = z_ref[...]
    m = jnp.max(z, axis=1, keepdims=True)
    lse = jnp.log(jnp.sum(jnp.exp(z - m), axis=1, keepdims=True)) + m
    o_ref[...] = z - lse


def _tc3b(z):
    return pl.pallas_call(
        _tc3b_body,
        grid=(GRID1,),
        in_specs=[pl.BlockSpec((ROWB, C), lambda i: (i, 0))],
        out_specs=pl.BlockSpec((ROWB, C), lambda i: (i, 0)),
        out_shape=jax.ShapeDtypeStruct((N, C), jnp.float32),
    )(z)


ERWS = E // CH
SRWS = NACC // CH
PADR = ROWS - ERWS - SRWS


def kernel(x, edge_index, W1, b1, W2, b2):
    ei = edge_index.astype(jnp.int32)
    s1 = ei[0].reshape(ERWS, CH)
    d1 = ei[1].reshape(ERWS, CH)
    selfs = jnp.arange(NACC, dtype=jnp.int32).reshape(SRWS, CH)
    pad_src = jnp.zeros((PADR, CH), jnp.int32)
    pad_dst = (jnp.arange(PADR * CH, dtype=jnp.int32) % 64 + N).reshape(
        PADR, CH)
    src2 = jnp.concatenate([s1, selfs, pad_src])
    dst2 = jnp.concatenate([d1, selfs, pad_dst])

    z16 = jnp.zeros((NACC, H), jnp.float32)
    z1 = jnp.zeros((NACC,), jnp.float32)

    h1 = _tc1(x, W1.T, b1.reshape(1, H))

    part1, cntp = _sc_pass_counts(src2, dst2, h1, z16, z1)
    cbp = jnp.repeat((cntp[:NACC] + cntp[NACC:]).reshape(NPK, 8), 16, axis=1)
    part1p = part1.reshape(NC, NPK, 128)

    gp = _tc2(part1p, cbp)

    part2 = _sc_pass(src2, dst2, gp.reshape(NACC, H), z16)

    w2d = jnp.kron(jnp.eye(8, dtype=jnp.float32), W2.T)
    b2d = jnp.tile(b2, 8).reshape(1, 8 * C)
    zp = _tc3a(part2.reshape(NC, NPK, 128), cbp, w2d, b2d)

    return _tc3b(zp.reshape(NACC, C))


# speedup vs baseline: 1.1140x; 1.1140x over previous
import functools

import jax
import jax.numpy as jnp
from jax import lax
from jax.experimental import pallas as pl
from jax.experimental.pallas import tpu as pltpu
from jax.experimental.pallas import tpu_sc as plsc

N = 100000
E = 3200000
F = 128
H = 16
C = 40

NC = 2
NS = 16
NW = NC * NS

CH = 128
RW = 816
ROWS = NW * RW
EPAD = ROWS * CH
K = 16
DEPTH = 4
NBUF = 8

NACC = 100352
RT = NACC // NS


def _sc_body(with_counts, *refs):
    if with_counts:
        (src_hbm, dst_hbm, tab_hbm, z16_hbm, z1_hbm,
         out_hbm, cnt_out_hbm,
         acc_sh, cnt_sh, sidx, didx, rows_v, ones_v, z1v,
         *sems) = refs
    else:
        (src_hbm, dst_hbm, tab_hbm, z16_hbm,
         out_hbm,
         acc_sh, sidx, didx, rows_v,
         *sems) = refs
    sem_g = sems[:NBUF]
    sem_s = sems[NBUF:2 * NBUF]
    sem_c = sems[2 * NBUF] if with_counts else None

    core = lax.axis_index("c")
    sub = lax.axis_index("s")
    w = sub * NC + core

    r0 = sub * RT

    def zinit(k, carry):
        o = r0 + k * CH
        pltpu.sync_copy(z16_hbm.at[pl.ds(o, CH)], rows_v.at[0])
        pltpu.sync_copy(rows_v.at[0], acc_sh.at[pl.ds(o, CH)])
        if with_counts:
            pltpu.sync_copy(z1_hbm.at[pl.ds(o, CH)], z1v)
            pltpu.sync_copy(z1v, cnt_sh.at[pl.ds(o, CH)])
        return carry

    lax.fori_loop(0, RT // CH, zinit, 0)
    if with_counts:
        for i in range(CH // 16):
            ones_v[pl.ds(i * 16, 16)] = jnp.ones((16,), jnp.float32)
    plsc.subcore_barrier()

    def step(i, carry):
        row = (i * NW + w) * K
        pltpu.sync_copy(src_hbm.at[pl.ds(row, K)], sidx)
        pltpu.sync_copy(dst_hbm.at[pl.ds(row, K)], didx)
        dg = {}
        dsc = {}
        dcnt = []
        for u in range(DEPTH):
            dg[u] = pltpu.async_copy(tab_hbm.at[sidx.at[u]], rows_v.at[u],
                                     sem_g[u])
        for u in range(K):
            dg[u].wait()
            dsc[u] = pltpu.async_copy(rows_v.at[u % NBUF],
                                      acc_sh.at[didx.at[u]],
                                      sem_s[u % NBUF], add=True)
            if with_counts:
                dcnt.append(pltpu.async_copy(ones_v, cnt_sh.at[didx.at[u]],
                                             sem_c, add=True))
            nu = u + DEPTH
            if nu < K:
                if nu >= NBUF:
                    dsc[nu - NBUF].wait()
                dg[nu] = pltpu.async_copy(tab_hbm.at[sidx.at[nu]],
                                          rows_v.at[nu % NBUF],
                                          sem_g[nu % NBUF])
        for u in range(K - NBUF, K):
            dsc[u].wait()
        for d in dcnt:
            d.wait()
        return carry

    lax.fori_loop(0, RW // K, step, 0)

    plsc.subcore_barrier()

    def cout(k, carry):
        o = r0 + k * CH
        pltpu.sync_copy(acc_sh.at[pl.ds(o, CH)], rows_v.at[0])
        pltpu.sync_copy(rows_v.at[0], out_hbm.at[core, pl.ds(o, CH)])
        if with_counts:
            pltpu.sync_copy(cnt_sh.at[pl.ds(o, CH)], z1v)
            pltpu.sync_copy(z1v, cnt_out_hbm.at[pl.ds(core * NACC + o, CH)])
        return carry

    lax.fori_loop(0, RT // CH, cout, 0)


def _make_sc_pass(with_counts):
    mesh = plsc.VectorSubcoreMesh(core_axis_name="c", subcore_axis_name="s",
                                  num_cores=NC, num_subcores=NS)
    if with_counts:
        out_type = (jax.ShapeDtypeStruct((NC, NACC, H), jnp.float32),
                    jax.ShapeDtypeStruct((NC * NACC,), jnp.float32))
        scratch = [
            pltpu.VMEM_SHARED((NACC, H), jnp.float32),
            pltpu.VMEM_SHARED((NACC,), jnp.float32),
            pltpu.VMEM((K, CH), jnp.int32),
            pltpu.VMEM((K, CH), jnp.int32),
            pltpu.VMEM((NBUF, CH, H), jnp.float32),
            pltpu.VMEM((CH,), jnp.float32),
            pltpu.VMEM((CH,), jnp.float32),
        ] + [pltpu.SemaphoreType.DMA] * (2 * NBUF + 1)
    else:
        out_type = jax.ShapeDtypeStruct((NC, NACC, H), jnp.float32)
        scratch = [
            pltpu.VMEM_SHARED((NACC, H), jnp.float32),
            pltpu.VMEM((K, CH), jnp.int32),
            pltpu.VMEM((K, CH), jnp.int32),
            pltpu.VMEM((NBUF, CH, H), jnp.float32),
        ] + [pltpu.SemaphoreType.DMA] * (2 * NBUF)
    return pl.kernel(functools.partial(_sc_body, with_counts),
                     out_type=out_type, mesh=mesh, scratch_types=scratch,
                     compiler_params=pltpu.CompilerParams(
                         use_tc_tiling_on_sc=False))


_sc_pass_counts = _make_sc_pass(True)
_sc_pass = _make_sc_pass(False)


NPK = NACC // 8
ROWB = 2048
GRID1 = NACC // ROWB
PKB = ROWB // 8


def _tc1_body(x_ref, w_ref, b_ref, o_ref):
    o_ref[...] = jnp.dot(x_ref[...], w_ref[...],
                         preferred_element_type=jnp.float32) + b_ref[...]


def _tc1(x, wt, b):
    return pl.pallas_call(
        _tc1_body,
        grid=(GRID1,),
        in_specs=[
            pl.BlockSpec((ROWB, F), lambda i: (i, 0)),
            pl.BlockSpec((F, H), lambda i: (0, 0)),
            pl.BlockSpec((1, H), lambda i: (0, 0)),
        ],
        out_specs=pl.BlockSpec((ROWB, H), lambda i: (i, 0)),
        out_shape=jax.ShapeDtypeStruct((NACC, H), jnp.float32),
    )(x, wt, b)


def _tc2_body(p_ref, c_ref, g_ref):
    p = p_ref[...]
    s = (p[0] + p[1]) / c_ref[...]
    g_ref[...] = jnp.maximum(s, 0.0)


def _tc2(partp, cbp):
    return pl.pallas_call(
        _tc2_body,
        grid=(GRID1,),
        in_specs=[
            pl.BlockSpec((NC, PKB, 128), lambda i: (0, i, 0)),
            pl.BlockSpec((PKB, 128), lambda i: (i, 0)),
        ],
        out_specs=pl.BlockSpec((PKB, 128), lambda i: (i, 0)),
        out_shape=jax.ShapeDtypeStruct((NPK, 128), jnp.float32),
    )(partp, cbp)


def _tc3a_body(p_ref, c_ref, w_ref, b_ref, o_ref):
    p = p_ref[...]
    sp = (p[0] + p[1]) / c_ref[...]
    o_ref[...] = jnp.dot(sp, w_ref[...],
                         preferred_element_type=jnp.float32) + b_ref[...]


def _tc3a(partp, cbp, wd, bd):
    return pl.pallas_call(
        _tc3a_body,
        grid=(GRID1,),
        in_specs=[
            pl.BlockSpec((NC, PKB, 128), lambda i: (0, i, 0)),
            pl.BlockSpec((PKB, 128), lambda i: (i, 0)),
            pl.BlockSpec((128, 8 * C), lambda i: (0, 0)),
            pl.BlockSpec((1, 8 * C), lambda i: (0, 0)),
        ],
        out_specs=pl.BlockSpec((PKB, 8 * C), lambda i: (i, 0)),
        out_shape=jax.ShapeDtypeStruct((NPK, 8 * C), jnp.float32),
    )(partp, cbp, wd, bd)


def _tc3b_body(z_ref, o_ref):
    z = z_ref[...]
    m = jnp.max(z, axis=1, keepdims=True)
    lse = jnp.log(jnp.sum(jnp.exp(z - m), axis=1, keepdims=True)) + m
    o_ref[...] = z - lse


def _tc3b(z):
    return pl.pallas_call(
        _tc3b_body,
        grid=(GRID1,),
        in_specs=[pl.BlockSpec((ROWB, C), lambda i: (i, 0))],
        out_specs=pl.BlockSpec((ROWB, C), lambda i: (i, 0)),
        out_shape=jax.ShapeDtypeStruct((N, C), jnp.float32),
    )(z)


ERWS = E // CH
SRWS = NACC // CH
PADR = ROWS - ERWS - SRWS


def kernel(x, edge_index, W1, b1, W2, b2):
    ei = edge_index.astype(jnp.int32)
    s1 = ei[0].reshape(ERWS, CH)
    d1 = ei[1].reshape(ERWS, CH)
    selfs = jnp.arange(NACC, dtype=jnp.int32).reshape(SRWS, CH)
    pad_src = jnp.zeros((PADR, CH), jnp.int32)
    pad_dst = (jnp.arange(PADR * CH, dtype=jnp.int32) % (NACC - N)
               + N).reshape(PADR, CH)
    src2 = jnp.concatenate([s1, selfs, pad_src])
    dst2 = jnp.concatenate([d1, selfs, pad_dst])

    z16 = jnp.zeros((NACC, H), jnp.float32)
    z1 = jnp.zeros((NACC,), jnp.float32)

    h1 = _tc1(x, W1.T, b1.reshape(1, H))

    part1, cntp = _sc_pass_counts(src2, dst2, h1, z16, z1)
    cbp = jnp.repeat((cntp[:NACC] + cntp[NACC:]).reshape(NPK, 8), 16, axis=1)
    part1p = part1.reshape(NC, NPK, 128)

    gp = _tc2(part1p, cbp)

    part2 = _sc_pass(src2, dst2, gp.reshape(NACC, H), z16)

    w2d = jnp.kron(jnp.eye(8, dtype=jnp.float32), W2.T)
    b2d = jnp.tile(b2, 8).reshape(1, 8 * C)
    zp = _tc3a(part2.reshape(NC, NPK, 128), cbp, w2d, b2d)

    return _tc3b(zp.reshape(NACC, C))


# speedup vs baseline: 1.2249x; 1.0995x over previous
import functools

import jax
import jax.numpy as jnp
from jax import lax
from jax.experimental import pallas as pl
from jax.experimental.pallas import tpu as pltpu
from jax.experimental.pallas import tpu_sc as plsc

N = 100000
E = 3200000
F = 128
H = 16
C = 40

NC = 2
NS = 16
NW = NC * NS

CH = 128
RW = 784
ROWS = NW * RW
EPAD = ROWS * CH
K = 16
DEPTH = 4
NBUF = 8

NACC = 100352
RT = NACC // NS


def _sc_body(with_counts, *refs):
    if with_counts:
        (src_hbm, dst_hbm, tab_hbm, z16_hbm, z1_hbm,
         out_hbm, cnt_out_hbm,
         acc_sh, cnt_sh, sidx, didx, rows_v, ones_v, z1v,
         *sems) = refs
    else:
        (src_hbm, dst_hbm, tab_hbm, z16_hbm,
         out_hbm,
         acc_sh, sidx, didx, rows_v,
         *sems) = refs
    sem_g = sems[:NBUF]
    sem_s = sems[NBUF:2 * NBUF]
    sem_c = sems[2 * NBUF] if with_counts else None

    core = lax.axis_index("c")
    sub = lax.axis_index("s")
    w = sub * NC + core

    r0 = sub * RT

    def zinit(k, carry):
        o = r0 + k * CH
        pltpu.sync_copy(z16_hbm.at[pl.ds(o, CH)], rows_v.at[0])
        pltpu.sync_copy(rows_v.at[0], acc_sh.at[pl.ds(o, CH)])
        if with_counts:
            pltpu.sync_copy(z1_hbm.at[pl.ds(o, CH)], z1v)
            pltpu.sync_copy(z1v, cnt_sh.at[pl.ds(o, CH)])
        return carry

    lax.fori_loop(0, RT // CH, zinit, 0)
    if with_counts:
        for i in range(CH // 16):
            ones_v[pl.ds(i * 16, 16)] = jnp.ones((16,), jnp.float32)
    plsc.subcore_barrier()

    def step(i, carry):
        row = (i * NW + w) * K
        pltpu.sync_copy(src_hbm.at[pl.ds(row, K)], sidx)
        pltpu.sync_copy(dst_hbm.at[pl.ds(row, K)], didx)
        dg = {}
        dsc = {}
        dcnt = []
        for u in range(DEPTH):
            dg[u] = pltpu.async_copy(tab_hbm.at[sidx.at[u]], rows_v.at[u],
                                     sem_g[u])
        for u in range(K):
            dg[u].wait()
            dsc[u] = pltpu.async_copy(rows_v.at[u % NBUF],
                                      acc_sh.at[didx.at[u]],
                                      sem_s[u % NBUF], add=True)
            if with_counts:
                dcnt.append(pltpu.async_copy(ones_v, cnt_sh.at[didx.at[u]],
                                             sem_c, add=True))
            nu = u + DEPTH
            if nu < K:
                if nu >= NBUF:
                    dsc[nu - NBUF].wait()
                dg[nu] = pltpu.async_copy(tab_hbm.at[sidx.at[nu]],
                                          rows_v.at[nu % NBUF],
                                          sem_g[nu % NBUF])
        for u in range(K - NBUF, K):
            dsc[u].wait()
        for d in dcnt:
            d.wait()
        return carry

    lax.fori_loop(0, RW // K, step, 0)

    plsc.subcore_barrier()

    def cout(k, carry):
        o = r0 + k * CH
        pltpu.sync_copy(acc_sh.at[pl.ds(o, CH)], rows_v.at[0])
        pltpu.sync_copy(rows_v.at[0], out_hbm.at[core, pl.ds(o, CH)])
        if with_counts:
            pltpu.sync_copy(cnt_sh.at[pl.ds(o, CH)], z1v)
            pltpu.sync_copy(z1v, cnt_out_hbm.at[pl.ds(core * NACC + o, CH)])
        return carry

    lax.fori_loop(0, RT // CH, cout, 0)


def _make_sc_pass(with_counts):
    mesh = plsc.VectorSubcoreMesh(core_axis_name="c", subcore_axis_name="s",
                                  num_cores=NC, num_subcores=NS)
    if with_counts:
        out_type = (jax.ShapeDtypeStruct((NC, NACC, H), jnp.float32),
                    jax.ShapeDtypeStruct((NC * NACC,), jnp.float32))
        scratch = [
            pltpu.VMEM_SHARED((NACC, H), jnp.float32),
            pltpu.VMEM_SHARED((NACC,), jnp.float32),
            pltpu.VMEM((K, CH), jnp.int32),
            pltpu.VMEM((K, CH), jnp.int32),
            pltpu.VMEM((NBUF, CH, H), jnp.float32),
            pltpu.VMEM((CH,), jnp.float32),
            pltpu.VMEM((CH,), jnp.float32),
        ] + [pltpu.SemaphoreType.DMA] * (2 * NBUF + 1)
    else:
        out_type = jax.ShapeDtypeStruct((NC, NACC, H), jnp.float32)
        scratch = [
            pltpu.VMEM_SHARED((NACC, H), jnp.float32),
            pltpu.VMEM((K, CH), jnp.int32),
            pltpu.VMEM((K, CH), jnp.int32),
            pltpu.VMEM((NBUF, CH, H), jnp.float32),
        ] + [pltpu.SemaphoreType.DMA] * (2 * NBUF)
    return pl.kernel(functools.partial(_sc_body, with_counts),
                     out_type=out_type, mesh=mesh, scratch_types=scratch,
                     compiler_params=pltpu.CompilerParams(
                         use_tc_tiling_on_sc=False))


_sc_pass_counts = _make_sc_pass(True)
_sc_pass = _make_sc_pass(False)


ROWB = 4000
GRID1 = N // ROWB
ROWB2 = 2048
GRID2 = NACC // ROWB2


def _tc1_body(x_ref, w_ref, b_ref, o_ref):
    o_ref[...] = jnp.dot(x_ref[...], w_ref[...],
                         preferred_element_type=jnp.float32) + b_ref[...]


def _tc1(x, wt, b):
    return pl.pallas_call(
        _tc1_body,
        grid=(GRID1,),
        in_specs=[
            pl.BlockSpec((ROWB, F), lambda i: (i, 0)),
            pl.BlockSpec((F, H), lambda i: (0, 0)),
            pl.BlockSpec((1, H), lambda i: (0, 0)),
        ],
        out_specs=pl.BlockSpec((ROWB, H), lambda i: (i, 0)),
        out_shape=jax.ShapeDtypeStruct((NACC, H), jnp.float32),
    )(x, wt, b)


def _tc2_body(p_ref, h_ref, c_ref, g_ref):
    p = p_ref[...]
    cnt = c_ref[...] + 1.0
    s = (p[0] + p[1] + h_ref[...]) / cnt
    g_ref[...] = jnp.maximum(s, 0.0)


def _tc2(part, h, cb):
    return pl.pallas_call(
        _tc2_body,
        grid=(GRID2,),
        in_specs=[
            pl.BlockSpec((NC, ROWB2, H), lambda i: (0, i, 0)),
            pl.BlockSpec((ROWB2, H), lambda i: (i, 0)),
            pl.BlockSpec((ROWB2, H), lambda i: (i, 0)),
        ],
        out_specs=pl.BlockSpec((ROWB2, H), lambda i: (i, 0)),
        out_shape=jax.ShapeDtypeStruct((NACC, H), jnp.float32),
    )(part, h, cb)


def _tc3_body(p_ref, g_ref, c_ref, w_ref, b_ref, o_ref):
    p = p_ref[...]
    s = (p[0] + p[1] + g_ref[...]) / (c_ref[...] + 1.0)
    z = jnp.dot(s, w_ref[...], preferred_element_type=jnp.float32) + b_ref[...]
    m = jnp.max(z, axis=1, keepdims=True)
    lse = jnp.log(jnp.sum(jnp.exp(z - m), axis=1, keepdims=True)) + m
    o_ref[...] = z - lse


def _tc3(part, g, cb, wt, b):
    return pl.pallas_call(
        _tc3_body,
        grid=(GRID2,),
        in_specs=[
            pl.BlockSpec((NC, ROWB2, H), lambda i: (0, i, 0)),
            pl.BlockSpec((ROWB2, H), lambda i: (i, 0)),
            pl.BlockSpec((ROWB2, H), lambda i: (i, 0)),
            pl.BlockSpec((H, C), lambda i: (0, 0)),
            pl.BlockSpec((1, C), lambda i: (0, 0)),
        ],
        out_specs=pl.BlockSpec((ROWB2, C), lambda i: (i, 0)),
        out_shape=jax.ShapeDtypeStruct((NACC, C), jnp.float32),
    )(part, g, cb, wt, b)


ERWS = E // CH
PADR = ROWS - ERWS


def kernel(x, edge_index, W1, b1, W2, b2):
    ei = edge_index.astype(jnp.int32)
    s1 = ei[0].reshape(ERWS, CH)
    d1 = ei[1].reshape(ERWS, CH)
    src2 = jnp.pad(s1, ((0, PADR), (0, 0)))
    pad_dst = (jnp.arange(PADR * CH, dtype=jnp.int32) % (NACC - N)
               + N).reshape(PADR, CH)
    dst2 = jnp.concatenate([d1, pad_dst])

    z16 = jnp.zeros((NACC, H), jnp.float32)
    z1 = jnp.zeros((NACC,), jnp.float32)

    h1 = _tc1(x, W1.T, b1.reshape(1, H))

    part1, cntp = _sc_pass_counts(src2, dst2, h1, z16, z1)
    cb = jnp.broadcast_to((cntp[:NACC] + cntp[NACC:])[:, None], (NACC, H))

    g = _tc2(part1, h1, cb)

    part2 = _sc_pass(src2, dst2, g, z16)

    return _tc3(part2, g, cb, W2.T, b2.reshape(1, C))[:N]
